# baseline (speedup 1.0000x reference)
import functools

import jax
import jax.numpy as jnp
from jax import lax
from jax.experimental import pallas as pl
from jax.experimental.pallas import tpu as pltpu
from jax.experimental.pallas import tpu_sc as plsc

NC = 2
NS = 16
NW = NC * NS
CH = 128


def _sc_mesh():
    return plsc.VectorSubcoreMesh(
        core_axis_name="c", subcore_axis_name="s", num_cores=NC, num_subcores=NS
    )


def _deg_call(dst2d, NPAD):
    nrows, ch = dst2d.shape
    nch = nrows // NW
    rpt = NPAD // NS

    @functools.partial(
        pl.kernel,
        out_type=jax.ShapeDtypeStruct((NC, NPAD), jnp.float32),
        mesh=_sc_mesh(),
        scratch_types=[
            pltpu.VMEM((nch, ch), jnp.int32),
            pltpu.VMEM((ch,), jnp.float32),
            pltpu.VMEM((rpt,), jnp.float32),
            pltpu.VMEM_SHARED((NPAD,), jnp.float32),
            pltpu.SemaphoreType.DMA,
        ],
    )
    def k(dst_hbm, out_hbm, dstv, onesv, zbuf, deg_sh, dsem):
        c = lax.axis_index("c")
        s = lax.axis_index("s")

        def zfill(i, _):
            zbuf[pl.ds(i * 16, 16)] = jnp.zeros((16,), jnp.float32)
            return 0

        lax.fori_loop(0, rpt // 16, zfill, 0)

        def ofill(i, _):
            onesv[pl.ds(i * 16, 16)] = jnp.ones((16,), jnp.float32)
            return 0

        lax.fori_loop(0, ch // 16, ofill, 0)

        pltpu.sync_copy(zbuf, deg_sh.at[pl.ds(s * rpt, rpt)])
        plsc.subcore_barrier()

        base = (c * NS + s) * nch
        pltpu.sync_copy(dst_hbm.at[pl.ds(base, nch)], dstv)

        def fire(j, _):
            pltpu.async_copy(onesv, deg_sh.at[dstv.at[j]], dsem, add=True)
            return 0

        lax.fori_loop(0, nch, fire, 0)

        def drain(j, _):
            pltpu.make_async_copy(onesv, deg_sh.at[dstv.at[j]], dsem).wait()
            return 0

        lax.fori_loop(0, nch, drain, 0)
        plsc.subcore_barrier()
        pltpu.sync_copy(deg_sh.at[pl.ds(s * rpt, rpt)], out_hbm.at[c, pl.ds(s * rpt, rpt)])

    return k(dst2d)


def _agg_call(src2d, dst2d, h2, NTAB):
    nrows, ch = src2d.shape
    N, D = h2.shape
    nch = nrows // NW
    rpt = NTAB // NS
    zr = 32

    NB = 16

    @functools.partial(
        pl.kernel,
        out_type=jax.ShapeDtypeStruct((NC, NTAB, D), jnp.float32),
        mesh=_sc_mesh(),
        scratch_types=[
            pltpu.VMEM((NB, ch), jnp.int32),
            pltpu.VMEM((NB, ch), jnp.int32),
            pltpu.VMEM((ch, D), jnp.float32),
            pltpu.VMEM((ch, D), jnp.float32),
            pltpu.VMEM((zr, D), jnp.float32),
            pltpu.VMEM_SHARED((NTAB, D), jnp.float32),
            pltpu.SemaphoreType.DMA,
            pltpu.SemaphoreType.DMA,
            pltpu.SemaphoreType.DMA,
            pltpu.SemaphoreType.DMA,
            pltpu.SemaphoreType.DMA,
        ],
    )
    def k(src_hbm, dst_hbm, h2_hbm, out_hbm, srcv, dstv, rows0, rows1, zbuf, agg_sh,
          gsem0, gsem1, ssem0, ssem1, zsem):
        c = lax.axis_index("c")
        s = lax.axis_index("s")

        def zfill(i, _):
            for q in range(D // 16):
                zbuf[i, pl.ds(q * 16, 16)] = jnp.zeros((16,), jnp.float32)
            return 0

        lax.fori_loop(0, zr, zfill, 0)
        for r in range(rpt // zr):
            pltpu.async_copy(zbuf, agg_sh.at[pl.ds(s * rpt + r * zr, zr)], zsem)
        for r in range(rpt // zr):
            pltpu.make_async_copy(zbuf, agg_sh.at[pl.ds(s * rpt + r * zr, zr)], zsem).wait()
        plsc.subcore_barrier()

        base = (c * NS + s) * nch

        hc = ch // 2

        def gfire(j, rows, sem):
            pltpu.async_copy(h2_hbm.at[srcv.at[j, pl.ds(0, hc)]], rows.at[pl.ds(0, hc)], sem)
            pltpu.async_copy(h2_hbm.at[srcv.at[j, pl.ds(hc, hc)]], rows.at[pl.ds(hc, hc)], sem)

        def gwait(j, rows, sem):
            pltpu.make_async_copy(h2_hbm.at[srcv.at[j, pl.ds(0, hc)]], rows.at[pl.ds(0, hc)], sem).wait()
            pltpu.make_async_copy(h2_hbm.at[srcv.at[j, pl.ds(hc, hc)]], rows.at[pl.ds(hc, hc)], sem).wait()

        def block(g, _):
            bb = base + g * NB
            pltpu.sync_copy(src_hbm.at[pl.ds(bb, NB)], srcv)
            pltpu.sync_copy(dst_hbm.at[pl.ds(bb, NB)], dstv)
            gfire(0, rows0, gsem0)

            def pair(p, _):
                j = 2 * p

                @pl.when(p > 0)
                def _():
                    pltpu.make_async_copy(rows1, agg_sh.at[dstv.at[j - 1]], ssem1).wait()

                gfire(j + 1, rows1, gsem1)
                gwait(j, rows0, gsem0)
                pltpu.async_copy(rows0, agg_sh.at[dstv.at[j]], ssem0, add=True)

                pltpu.make_async_copy(rows0, agg_sh.at[dstv.at[j]], ssem0).wait()

                @pl.when(j + 2 < NB)
                def _():
                    gfire(j + 2, rows0, gsem0)

                gwait(j + 1, rows1, gsem1)
                pltpu.async_copy(rows1, agg_sh.at[dstv.at[j + 1]], ssem1, add=True)
                return 0

            lax.fori_loop(0, NB // 2, pair, 0)
            pltpu.make_async_copy(rows1, agg_sh.at[dstv.at[NB - 1]], ssem1).wait()
            return 0

        lax.fori_loop(0, nch // NB, block, 0)
        plsc.subcore_barrier()
        pltpu.sync_copy(agg_sh.at[pl.ds(s * rpt, rpt)], out_hbm.at[c, pl.ds(s * rpt, rpt)])

    return k(src2d, dst2d, h2)


def _edges_call(edge_index, NROWS, N, NTAB):
    _, E = edge_index.shape
    n0 = E // CH
    npr = NROWS - n0

    def body(e_ref, src_ref, dst_ref):
        src_ref[0:n0, :] = jnp.reshape(e_ref[0:1, :], (n0, CH))
        dst_ref[0:n0, :] = jnp.reshape(e_ref[1:2, :], (n0, CH))
        k = (lax.broadcasted_iota(jnp.int32, (npr, CH), 0) * CH
             + lax.broadcasted_iota(jnp.int32, (npr, CH), 1))
        src_ref[n0:NROWS, :] = k % N
        dst_ref[n0:NROWS, :] = N + k % (NTAB - N)

    return pl.pallas_call(
        body,
        out_shape=(
            jax.ShapeDtypeStruct((NROWS, CH), jnp.int32),
            jax.ShapeDtypeStruct((NROWS, CH), jnp.int32),
        ),
    )(edge_index)


def _h2_call(x, W, degp):
    N, DI = x.shape
    DO = W.shape[1]

    def body(x_ref, w_ref, degp_ref, h2_ref, dis_ref):
        h = jnp.dot(x_ref[...], w_ref[...], preferred_element_type=jnp.float32)
        deg_row = degp_ref[0:1, :] + degp_ref[1:2, :] + 1.0
        dis_col = jnp.transpose(lax.rsqrt(deg_row))[0:N, :]
        h2_ref[...] = h * dis_col
        dis_ref[...] = dis_col

    return pl.pallas_call(
        body,
        out_shape=(
            jax.ShapeDtypeStruct((N, DO), jnp.float32),
            jax.ShapeDtypeStruct((N, 1), jnp.float32),
        ),
    )(x, W, degp)


def _final_call(aggp, h2, dis, b2, g2, be2):
    N, D = h2.shape

    def body(aggp_ref, h2_ref, dis_ref, b_ref, g_ref, be_ref, out_ref):
        a0 = aggp_ref[0, 0:N, :]
        a1 = aggp_ref[1, 0:N, :]
        y = (a0 + a1 + h2_ref[...]) * dis_ref[...] + b_ref[...]
        n = jnp.float32(N)
        mean = jnp.sum(y, axis=0, keepdims=True) / n
        var = jnp.sum(y * y, axis=0, keepdims=True) / n - mean * mean
        inv = lax.rsqrt(var + 1e-5)
        out_ref[...] = jnp.maximum((y - mean) * inv * g_ref[...] + be_ref[...], 0.0)

    return pl.pallas_call(
        body,
        out_shape=jax.ShapeDtypeStruct((N, D), jnp.float32),
    )(aggp, h2, dis, b2, g2, be2)


def kernel(x, edge_index, W, b, gamma, beta):
    N, DI = x.shape
    DO = W.shape[1]
    E = edge_index.shape[1]
    NPAD = ((N + NS * 16 - 1) // (NS * 16)) * NS * 16
    NTAB = NPAD

    nch = -(-E // (NW * CH))
    nch = ((nch + 7) // 8) * 8
    src2d, dst2d = _edges_call(edge_index, NW * nch, N, NTAB)

    degp = _deg_call(dst2d, NPAD)
    h2, dis = _h2_call(x, W, degp)
    aggp = _agg_call(src2d, dst2d, h2, NTAB)
    return _final_call(
        aggp, h2, dis,
        b.reshape(1, DO), gamma.reshape(1, DO), beta.reshape(1, DO),
    )


# speedup vs baseline: 1.0230x; 1.0230x over previous
import functools

import jax
import jax.numpy as jnp
from jax import lax
from jax.experimental import pallas as pl
from jax.experimental.pallas import tpu as pltpu
from jax.experimental.pallas import tpu_sc as plsc

NC = 2
NS = 16
NW = NC * NS
CH = 128


def _sc_mesh():
    return plsc.VectorSubcoreMesh(
        core_axis_name="c", subcore_axis_name="s", num_cores=NC, num_subcores=NS
    )


def _deg_call(dst2d, NPAD):
    nrows, ch = dst2d.shape
    nch = nrows // NW
    rpt = NPAD // NS

    @functools.partial(
        pl.kernel,
        out_type=jax.ShapeDtypeStruct((NC, NPAD), jnp.float32),
        mesh=_sc_mesh(),
        scratch_types=[
            pltpu.VMEM((nch, ch), jnp.int32),
            pltpu.VMEM((ch,), jnp.float32),
            pltpu.VMEM((rpt,), jnp.float32),
            pltpu.VMEM_SHARED((NPAD,), jnp.float32),
            pltpu.SemaphoreType.DMA,
        ],
    )
    def k(dst_hbm, out_hbm, dstv, onesv, zbuf, deg_sh, dsem):
        c = lax.axis_index("c")
        s = lax.axis_index("s")

        def zfill(i, _):
            zbuf[pl.ds(i * 16, 16)] = jnp.zeros((16,), jnp.float32)
            return 0

        lax.fori_loop(0, rpt // 16, zfill, 0)

        def ofill(i, _):
            onesv[pl.ds(i * 16, 16)] = jnp.ones((16,), jnp.float32)
            return 0

        lax.fori_loop(0, ch // 16, ofill, 0)

        pltpu.sync_copy(zbuf, deg_sh.at[pl.ds(s * rpt, rpt)])
        plsc.subcore_barrier()

        base = (c * NS + s) * nch
        pltpu.sync_copy(dst_hbm.at[pl.ds(base, nch)], dstv)

        def fire(j, _):
            pltpu.async_copy(onesv, deg_sh.at[dstv.at[j]], dsem, add=True)
            return 0

        lax.fori_loop(0, nch, fire, 0)

        def drain(j, _):
            pltpu.make_async_copy(onesv, deg_sh.at[dstv.at[j]], dsem).wait()
            return 0

        lax.fori_loop(0, nch, drain, 0)
        plsc.subcore_barrier()
        pltpu.sync_copy(deg_sh.at[pl.ds(s * rpt, rpt)], out_hbm.at[c, pl.ds(s * rpt, rpt)])

    return k(dst2d)


def _agg_call(src2d, dst2d, h2, NTAB):
    nrows, ch = src2d.shape
    N, D = h2.shape
    nch = nrows // NW
    rpt = NTAB // NS
    zr = 32

    NB = 16

    @functools.partial(
        pl.kernel,
        out_type=jax.ShapeDtypeStruct((NC, NTAB, D), jnp.float32),
        mesh=_sc_mesh(),
        scratch_types=[
            pltpu.VMEM((NB, ch), jnp.int32),
            pltpu.VMEM((NB, ch), jnp.int32),
            pltpu.VMEM((ch, D), jnp.float32),
            pltpu.VMEM((ch, D), jnp.float32),
            pltpu.VMEM((zr, D), jnp.float32),
            pltpu.VMEM_SHARED((NTAB, D), jnp.float32),
            pltpu.SemaphoreType.DMA,
            pltpu.SemaphoreType.DMA,
            pltpu.SemaphoreType.DMA,
            pltpu.SemaphoreType.DMA,
            pltpu.SemaphoreType.DMA,
        ],
    )
    def k(src_hbm, dst_hbm, h2_hbm, out_hbm, srcv, dstv, rows0, rows1, zbuf, agg_sh,
          gsem0, gsem1, ssem0, ssem1, zsem):
        c = lax.axis_index("c")
        s = lax.axis_index("s")

        def zfill(i, _):
            for q in range(D // 16):
                zbuf[i, pl.ds(q * 16, 16)] = jnp.zeros((16,), jnp.float32)
            return 0

        lax.fori_loop(0, zr, zfill, 0)
        base = (c * NS + s) * nch
        for r in range(rpt // zr):
            pltpu.async_copy(zbuf, agg_sh.at[pl.ds(s * rpt + r * zr, zr)], zsem)
        pltpu.sync_copy(src_hbm.at[pl.ds(base, NB)], srcv)
        pltpu.sync_copy(dst_hbm.at[pl.ds(base, NB)], dstv)
        pltpu.async_copy(h2_hbm.at[srcv.at[0]], rows0, gsem0)
        for r in range(rpt // zr):
            pltpu.make_async_copy(zbuf, agg_sh.at[pl.ds(s * rpt + r * zr, zr)], zsem).wait()
        plsc.subcore_barrier()

        def block(g, _):
            bb = base + g * NB

            @pl.when(g > 0)
            def _():
                pltpu.sync_copy(src_hbm.at[pl.ds(bb, NB)], srcv)
                pltpu.sync_copy(dst_hbm.at[pl.ds(bb, NB)], dstv)
                pltpu.async_copy(h2_hbm.at[srcv.at[0]], rows0, gsem0)

            def pair(p, _):
                j = 2 * p

                @pl.when(p > 0)
                def _():
                    pltpu.make_async_copy(rows1, agg_sh.at[dstv.at[j - 1]], ssem1).wait()

                pltpu.async_copy(h2_hbm.at[srcv.at[j + 1]], rows1, gsem1)
                pltpu.make_async_copy(h2_hbm.at[srcv.at[j]], rows0, gsem0).wait()
                pltpu.async_copy(rows0, agg_sh.at[dstv.at[j]], ssem0, add=True)

                pltpu.make_async_copy(rows0, agg_sh.at[dstv.at[j]], ssem0).wait()

                @pl.when(j + 2 < NB)
                def _():
                    pltpu.async_copy(h2_hbm.at[srcv.at[j + 2]], rows0, gsem0)

                pltpu.make_async_copy(h2_hbm.at[srcv.at[j + 1]], rows1, gsem1).wait()
                pltpu.async_copy(rows1, agg_sh.at[dstv.at[j + 1]], ssem1, add=True)
                return 0

            lax.fori_loop(0, NB // 2, pair, 0)
            pltpu.make_async_copy(rows1, agg_sh.at[dstv.at[NB - 1]], ssem1).wait()
            return 0

        lax.fori_loop(0, nch // NB, block, 0)
        plsc.subcore_barrier()
        pltpu.sync_copy(agg_sh.at[pl.ds(s * rpt, rpt)], out_hbm.at[c, pl.ds(s * rpt, rpt)])

    return k(src2d, dst2d, h2)


def _edges_call(edge_index, NROWS, N, NTAB):
    _, E = edge_index.shape
    n0 = E // CH
    npr = NROWS - n0

    def body(e_ref, src_ref, dst_ref):
        src_ref[0:n0, :] = jnp.reshape(e_ref[0:1, :], (n0, CH))
        dst_ref[0:n0, :] = jnp.reshape(e_ref[1:2, :], (n0, CH))
        k = (lax.broadcasted_iota(jnp.int32, (npr, CH), 0) * CH
             + lax.broadcasted_iota(jnp.int32, (npr, CH), 1))
        src_ref[n0:NROWS, :] = k % N
        dst_ref[n0:NROWS, :] = N + k % (NTAB - N)

    return pl.pallas_call(
        body,
        out_shape=(
            jax.ShapeDtypeStruct((NROWS, CH), jnp.int32),
            jax.ShapeDtypeStruct((NROWS, CH), jnp.int32),
        ),
    )(edge_index)


def _h2_call(x, W, degp):
    N, DI = x.shape
    DO = W.shape[1]

    def body(x_ref, w_ref, degp_ref, h2_ref, dis_ref):
        h = jnp.dot(x_ref[...], w_ref[...], preferred_element_type=jnp.float32)
        deg_row = degp_ref[0:1, :] + degp_ref[1:2, :] + 1.0
        dis_col = jnp.transpose(lax.rsqrt(deg_row))[0:N, :]
        h2_ref[...] = h * dis_col
        dis_ref[...] = dis_col

    return pl.pallas_call(
        body,
        out_shape=(
            jax.ShapeDtypeStruct((N, DO), jnp.float32),
            jax.ShapeDtypeStruct((N, 1), jnp.float32),
        ),
    )(x, W, degp)


def _final_call(aggp, h2, dis, b2, g2, be2):
    N, D = h2.shape

    def body(aggp_ref, h2_ref, dis_ref, b_ref, g_ref, be_ref, out_ref):
        a0 = aggp_ref[0, 0:N, :]
        a1 = aggp_ref[1, 0:N, :]
        y = (a0 + a1 + h2_ref[...]) * dis_ref[...] + b_ref[...]
        n = jnp.float32(N)
        mean = jnp.sum(y, axis=0, keepdims=True) / n
        var = jnp.sum(y * y, axis=0, keepdims=True) / n - mean * mean
        inv = lax.rsqrt(var + 1e-5)
        out_ref[...] = jnp.maximum((y - mean) * inv * g_ref[...] + be_ref[...], 0.0)

    return pl.pallas_call(
        body,
        out_shape=jax.ShapeDtypeStruct((N, D), jnp.float32),
    )(aggp, h2, dis, b2, g2, be2)


def kernel(x, edge_index, W, b, gamma, beta):
    N, DI = x.shape
    DO = W.shape[1]
    E = edge_index.shape[1]
    NPAD = ((N + NS * 16 - 1) // (NS * 16)) * NS * 16
    NTAB = NPAD

    nch = -(-E // (NW * CH))
    nch = ((nch + 7) // 8) * 8
    src2d, dst2d = _edges_call(edge_index, NW * nch, N, NTAB)

    degp = _deg_call(dst2d, NPAD)
    h2, dis = _h2_call(x, W, degp)
    aggp = _agg_call(src2d, dst2d, h2, NTAB)
    return _final_call(
        aggp, h2, dis,
        b.reshape(1, DO), gamma.reshape(1, DO), beta.reshape(1, DO),
    )
